# baseline (speedup 1.0000x reference)
import jax
import jax.numpy as jnp
from jax import lax
from jax.experimental import pallas as pl
from jax.experimental.pallas import tpu as pltpu

_T = 128


def _body(x_ref, o_ref, carry_ref):
    s = pl.program_id(1)

    @pl.when(s == 0)
    def _():
        carry_ref[...] = jnp.zeros_like(carry_ref)

    x = x_ref[0]
    t = x.shape[0]
    row = lax.broadcasted_iota(jnp.int32, (t, t), 0)
    col = lax.broadcasted_iota(jnp.int32, (t, t), 1)
    tri = (row >= col).astype(jnp.float32)
    local = jnp.dot(tri, x, preferred_element_type=jnp.float32)
    out = local + carry_ref[...]
    o_ref[0] = out
    carry_ref[...] = out[t - 1 :, :]


def kernel(input):
    b, s, d = input.shape
    grid = (b, s // _T)
    return pl.pallas_call(
        _body,
        grid=grid,
        in_specs=[pl.BlockSpec((1, _T, d), lambda i, j: (i, j, 0))],
        out_specs=pl.BlockSpec((1, _T, d), lambda i, j: (i, j, 0)),
        out_shape=jax.ShapeDtypeStruct((b, s, d), input.dtype),
        scratch_shapes=[pltpu.VMEM((1, d), jnp.float32)],
        compiler_params=pltpu.CompilerParams(
            dimension_semantics=("arbitrary", "arbitrary"),
        ),
    )(input)


# speedup vs baseline: 1.5740x; 1.5740x over previous
import jax
import jax.numpy as jnp
from jax import lax
from jax.experimental import pallas as pl
from jax.experimental.pallas import tpu as pltpu

_T = 512


def _body(x_ref, o_ref, carry_ref):
    s = pl.program_id(1)

    @pl.when(s == 0)
    def _():
        carry_ref[...] = jnp.zeros_like(carry_ref)

    x = x_ref[0]
    t = x.shape[0]
    row = lax.broadcasted_iota(jnp.int32, (t, t), 0)
    col = lax.broadcasted_iota(jnp.int32, (t, t), 1)
    tri = (row >= col).astype(jnp.float32)
    local = jnp.dot(tri, x, preferred_element_type=jnp.float32)
    out = local + carry_ref[...]
    o_ref[0] = out
    carry_ref[...] = out[t - 1 :, :]


def kernel(input):
    b, s, d = input.shape
    grid = (b, s // _T)
    return pl.pallas_call(
        _body,
        grid=grid,
        in_specs=[pl.BlockSpec((1, _T, d), lambda i, j: (i, j, 0))],
        out_specs=pl.BlockSpec((1, _T, d), lambda i, j: (i, j, 0)),
        out_shape=jax.ShapeDtypeStruct((b, s, d), input.dtype),
        scratch_shapes=[pltpu.VMEM((1, d), jnp.float32)],
        compiler_params=pltpu.CompilerParams(
            dimension_semantics=("arbitrary", "arbitrary"),
        ),
    )(input)
